# baseline (speedup 1.0000x reference)
import jax
import jax.numpy as jnp
from jax.experimental import pallas as pl
from jax.experimental.pallas import tpu as pltpu

_PRED_T = 0.3
_TRUE_T = 0.5
_PW = 0.05
_HB = 56


def _bbox_body(p_ref, e_ref, out_ref, yb, psum_ref):
    b = pl.program_id(0)
    h = pl.program_id(1)
    nb = pl.num_programs(0)
    nh = pl.num_programs(1)
    HB, W = p_ref.shape[1], p_ref.shape[2]
    H = HB * nh

    pm = jnp.max(p_ref[0], axis=2)
    em = jnp.max(e_ref[0], axis=2)

    hidx = jax.lax.broadcasted_iota(jnp.int32, (HB, W), 0) + h * HB
    widx = jax.lax.broadcasted_iota(jnp.int32, (HB, W), 1)

    pmask = pm > _PRED_T
    emask = em > _TRUE_T

    first = h == 0
    pymin = jnp.min(jnp.where(pmask, hidx, H))
    pymax = jnp.max(jnp.where(pmask, hidx, -1))
    pxmin = jnp.min(jnp.where(pmask, widx, W))
    pxmax = jnp.max(jnp.where(pmask, widx, -1))
    tymin = jnp.min(jnp.where(emask, hidx, H))
    tymax = jnp.max(jnp.where(emask, hidx, -1))
    txmin = jnp.min(jnp.where(emask, widx, W))
    txmax = jnp.max(jnp.where(emask, widx, -1))

    yb[0] = jnp.minimum(jnp.where(first, H, yb[0]), pymin)
    yb[1] = jnp.maximum(jnp.where(first, -1, yb[1]), pymax)
    yb[2] = jnp.minimum(jnp.where(first, W, yb[2]), pxmin)
    yb[3] = jnp.maximum(jnp.where(first, -1, yb[3]), pxmax)
    yb[4] = jnp.minimum(jnp.where(first, H, yb[4]), tymin)
    yb[5] = jnp.maximum(jnp.where(first, -1, yb[5]), tymax)
    yb[6] = jnp.minimum(jnp.where(first, W, yb[6]), txmin)
    yb[7] = jnp.maximum(jnp.where(first, -1, yb[7]), txmax)

    @pl.when(h == nh - 1)
    def _tail():
        f32 = jnp.float32

        def vec(s):
            return jnp.full((1, 128), s, f32)

        py1, py2, px1, px2 = yb[0], yb[1], yb[2], yb[3]
        ty1, ty2, tx1, tx2 = yb[4], yb[5], yb[6], yb[7]
        pa = vec((py2 - py1 + 1) * (px2 - px1 + 1))
        ta = vec((ty2 - ty1 + 1) * (tx2 - tx1 + 1))
        area_pen = jnp.maximum(pa - ta, 0.0) / (ta + 1.0)
        cy = vec(py1 + py2) * 0.5 - vec(ty1 + ty2) * 0.5
        cx = vec(px1 + px2) * 0.5 - vec(tx1 + tx2) * 0.5
        center = jnp.sqrt(cy * cy + cx * cx) * (1.0 / 20.0)
        valid = jnp.full((1, 128), (py2 >= 0) & (ty2 >= 0), jnp.bool_)
        pen = jnp.where(valid, area_pen + center, 1.0)
        prev = jnp.where(b == 0, jnp.zeros_like(pen), psum_ref[...])
        tot = prev + pen
        psum_ref[...] = tot

        @pl.when(b == nb - 1)
        def _():
            out_ref[...] = tot * (_PW / nb)


def kernel(prediction_probs, expected_onehot):
    B, H, W, C = prediction_probs.shape
    nh = H // _HB
    out = pl.pallas_call(
        _bbox_body,
        grid=(B, nh),
        in_specs=[
            pl.BlockSpec((1, _HB, W, C), lambda b, h: (b, h, 0, 0)),
            pl.BlockSpec((1, _HB, W, C), lambda b, h: (b, h, 0, 0)),
        ],
        out_specs=pl.BlockSpec((1, 128), lambda b, h: (0, 0)),
        out_shape=jax.ShapeDtypeStruct((1, 128), jnp.float32),
        scratch_shapes=[
            pltpu.SMEM((8,), jnp.int32),
            pltpu.VMEM((1, 128), jnp.float32),
        ],
    )(prediction_probs, expected_onehot)
    return out[0, 0]
